# baseline (speedup 1.0000x reference)
import functools

import jax
import jax.numpy as jnp
from jax import lax
from jax.experimental import pallas as pl
from jax.experimental.pallas import tpu as pltpu
from jax.experimental.pallas import tpu_sc as plsc

NC = 2
NS = 16
NW = NC * NS
CHUNK = 128


NB = 4
PF = 2
D = 64


def _make_spmm(n, nch, nparts):
    d = D
    epw = nch * CHUNK
    npt = n // NS
    assert n % NS == 0 and npt % CHUNK == 0 and nch % NB == 0 and nch >= 2 * NB
    pieces = npt // CHUNK
    nouter = nch // NB

    def body(*args):
        h_parts = args[:nparts]
        (row_hbm, col_hbm, adj_hbm, out_hbm,
         idx_row, idx_col, adjv, rows_v, acc_sh) = args[nparts:nparts + 9]
        sems = args[nparts + 9:]
        gsem = sems[:NB]
        ssem = sems[NB:2 * NB]
        asem = sems[2 * NB:]
        c = lax.axis_index("c")
        s = lax.axis_index("s")
        wid = c * NS + s

        def slot(b):
            return rows_v.at[pl.ds(b * CHUNK, CHUNK)]

        def gather_start(h_hbm, ch, b):
            pltpu.async_copy(h_hbm.at[idx_col.at[pl.ds(ch * CHUNK, CHUNK)]],
                             slot(b), gsem[b])
            pltpu.async_copy(adj_hbm.at[wid, pl.ds(ch * CHUNK, CHUNK)],
                             adjv.at[pl.ds(b * CHUNK, CHUNK)], asem[b])

        def gather_wait(h_hbm, b):
            pltpu.make_async_copy(h_hbm.at[pl.ds(0, CHUNK)], slot(b),
                                  gsem[b]).wait()
            pltpu.make_async_copy(adj_hbm.at[wid, pl.ds(0, CHUNK)],
                                  adjv.at[pl.ds(b * CHUNK, CHUNK)],
                                  asem[b]).wait()

        def scatter_start(ch, b):
            pltpu.async_copy(slot(b), acc_sh.at[idx_row.at[ch]], ssem[b],
                             add=True)

        def scatter_wait(b):
            pltpu.make_async_copy(slot(b), acc_sh.at[pl.ds(0, CHUNK)],
                                  ssem[b]).wait()

        def scale(ch, b):
            @pl.loop(0, CHUNK, unroll=8)
            def _scale(e):
                r = b * CHUNK + e
                av = adjv[r, :]
                for j in range(d // 16):
                    sl = pl.ds(j * 16, 16)
                    rows_v[r, sl] = rows_v[r, sl] * av

        pltpu.sync_copy(row_hbm.at[wid], idx_row)
        pltpu.sync_copy(col_hbm.at[wid], idx_col)

        base = s * npt

        for part in range(nparts):
            h_hbm = h_parts[part]

            def work(ch, b):
                gather_wait(h_hbm, b)
                scale(ch, b)
                scatter_start(ch, b)

            @pl.loop(0, CHUNK)
            def _zero(r):
                for j in range(d // 16):
                    rows_v[r, pl.ds(j * 16, 16)] = jnp.zeros((16,),
                                                             jnp.float32)

            for k in range(pieces):
                pltpu.sync_copy(rows_v.at[pl.ds(0, CHUNK)],
                                acc_sh.at[pl.ds(base + k * CHUNK, CHUNK)])
            plsc.subcore_barrier()

            for b in range(PF):
                gather_start(h_hbm, b, b)
            for b in range(NB):
                tgt = b + PF
                if tgt >= NB:
                    scatter_wait(tgt % NB)
                gather_start(h_hbm, tgt, tgt % NB)
                work(b, b)

            @pl.loop(1, nouter - 1)
            def _groups(g0):
                for b in range(NB):
                    ch = g0 * NB + b
                    scatter_wait((b + PF) % NB)
                    gather_start(h_hbm, ch + PF, (b + PF) % NB)
                    work(ch, b)

            for b in range(NB):
                ch = nch - NB + b
                if b < PF:
                    scatter_wait((b + PF) % NB)
                    gather_start(h_hbm, ch + PF, (b + PF) % NB)
                work(ch, b)
            for b in range(NB):
                scatter_wait(b)

            plsc.subcore_barrier()
            for k in range(pieces):
                pltpu.sync_copy(acc_sh.at[pl.ds(base + k * CHUNK, CHUNK)],
                                out_hbm.at[c, part,
                                           pl.ds(base + k * CHUNK, CHUNK)])
            if part + 1 < nparts:
                plsc.subcore_barrier()

    return pl.kernel(
        body,
        out_type=jax.ShapeDtypeStruct((NC, nparts, n, d), jnp.float32),
        compiler_params=pltpu.CompilerParams(use_tc_tiling_on_sc=False),
        mesh=plsc.VectorSubcoreMesh(core_axis_name="c", subcore_axis_name="s"),
        scratch_types=[
            pltpu.VMEM((nch, CHUNK), jnp.int32),
            pltpu.VMEM((epw,), jnp.int32),
            pltpu.VMEM((NB * CHUNK, 16), jnp.float32),
            pltpu.VMEM((NB * CHUNK, d), jnp.float32),
            pltpu.VMEM_SHARED((n, d), jnp.float32),
        ] + [pltpu.SemaphoreType.DMA] * (3 * NB),
    )


def _mm1(x, w):
    n, kdim = x.shape
    bm = 1000

    def kern(x_ref, w_ref, lo_ref, hi_ref):
        h = jnp.dot(x_ref[...], w_ref[...],
                    preferred_element_type=jnp.float32)
        lo_ref[...] = h[:, :D]
        hi_ref[...] = h[:, D:]

    return pl.pallas_call(
        kern,
        grid=(n // bm,),
        in_specs=[pl.BlockSpec((bm, kdim), lambda i: (i, 0)),
                  pl.BlockSpec((kdim, 2 * D), lambda i: (0, 0))],
        out_specs=[pl.BlockSpec((bm, D), lambda i: (i, 0)),
                   pl.BlockSpec((bm, D), lambda i: (i, 0))],
        out_shape=[jax.ShapeDtypeStruct((n, D), jnp.float32),
                   jax.ShapeDtypeStruct((n, D), jnp.float32)],
    )(x, w)


def _mid(p0lo, p1lo, p0hi, p1hi, b1, w2):
    n = p0lo.shape[0]
    kdim = w2.shape[0]
    m = w2.shape[1]
    bm = 1000

    def kern(a_ref, b_ref, c_ref, d_ref, b1_ref, w_ref, o_ref):
        lo = a_ref[...] + b_ref[...]
        hi = c_ref[...] + d_ref[...]
        a = jnp.maximum(jnp.concatenate([lo, hi], axis=1) + b1_ref[...], 0.0)
        o_ref[...] = jnp.dot(a, w_ref[...], preferred_element_type=jnp.float32)

    part = pl.BlockSpec((bm, D), lambda i: (i, 0))
    return pl.pallas_call(
        kern,
        grid=(n // bm,),
        in_specs=[part, part, part, part,
                  pl.BlockSpec((1, kdim), lambda i: (0, 0)),
                  pl.BlockSpec((kdim, m), lambda i: (0, 0))],
        out_specs=pl.BlockSpec((bm, m), lambda i: (i, 0)),
        out_shape=jax.ShapeDtypeStruct((n, m), jnp.float32),
    )(p0lo, p1lo, p0hi, p1hi, b1.reshape(1, kdim), w2)


def _post(q0, q1, b2p, nclass):
    n, dp = q0.shape
    bm = 1000

    def kern(q0_ref, q1_ref, b_ref, o_ref):
        z = q0_ref[...] + q1_ref[...] + b_ref[...]
        mask = lax.broadcasted_iota(jnp.int32, z.shape, 1) < nclass
        zm = jnp.where(mask, z, -jnp.inf)
        m = jnp.max(zm, axis=1, keepdims=True)
        ez = jnp.where(mask, jnp.exp(z - m), 0.0)
        lse = jnp.log(jnp.sum(ez, axis=1, keepdims=True))
        o_ref[...] = (z - m - lse)[:, :nclass]

    return pl.pallas_call(
        kern,
        grid=(n // bm,),
        in_specs=[pl.BlockSpec((bm, dp), lambda i: (i, 0)),
                  pl.BlockSpec((bm, dp), lambda i: (i, 0)),
                  pl.BlockSpec((1, dp), lambda i: (0, 0))],
        out_specs=pl.BlockSpec((bm, nclass), lambda i: (i, 0)),
        out_shape=jax.ShapeDtypeStruct((n, nclass), jnp.float32),
    )(q0, q1, b2p.reshape(1, dp))


def kernel(x, edge_index, adj_values, W1, b1, W2, b2):
    n, nfeat = x.shape
    e = edge_index.shape[1]
    nhid = W1.shape[1]
    nclass = W2.shape[1]
    d2 = 64

    nch = -(-e // (NW * CHUNK))
    nch = -(-nch // NB) * NB
    nch = max(nch, 2 * NB)
    ep = NW * nch * CHUNK
    row = jnp.pad(edge_index[0], (0, ep - e)).reshape(NW, nch, CHUNK)
    col = jnp.pad(edge_index[1], (0, ep - e)).reshape(NW, nch * CHUNK)
    adjp = jnp.pad(adj_values, (0, ep - e))
    adj = jnp.broadcast_to(adjp[:, None], (ep, 16)).reshape(NW, nch * CHUNK,
                                                            16)

    w2p = jnp.pad(W2, ((0, 0), (0, d2 - nclass)))
    b2p = jnp.pad(b2, (0, d2 - nclass))

    npad = -(-n // (NS * CHUNK)) * NS * CHUNK

    h_lo, h_hi = _mm1(x, W1)
    p = _make_spmm(npad, nch, 2)(h_lo, h_hi, row, col, adj)
    h2 = _mid(p[0, 0, :n], p[1, 0, :n], p[0, 1, :n], p[1, 1, :n],
              b1, w2p)
    q = _make_spmm(npad, nch, 1)(h2, row, col, adj)
    return _post(q[0, 0, :n], q[1, 0, :n], b2p, nclass)


# speedup vs baseline: 1.1760x; 1.1760x over previous
import functools

import jax
import jax.numpy as jnp
from jax import lax
from jax.experimental import pallas as pl
from jax.experimental.pallas import tpu as pltpu
from jax.experimental.pallas import tpu_sc as plsc

NC = 2
NS = 16
NW = NC * NS
CHUNK = 128


NB = 5
PF = 3
D = 64


def _make_spmm(n, nch, nparts):
    d = D
    epw = nch * CHUNK
    npt = n // NS
    assert n % NS == 0 and npt % CHUNK == 0 and nch % NB == 0 and nch >= 2 * NB
    pieces = npt // CHUNK
    nouter = nch // NB

    def body(*args):
        h_parts = args[:nparts]
        (row_hbm, col_hbm, adj_hbm, out_hbm,
         idx_row, idx_col, adj_all, rows_v, acc_sh) = args[nparts:nparts + 9]
        sems = args[nparts + 9:]
        gsem = sems[:NB]
        ssem = sems[NB:2 * NB]
        c = lax.axis_index("c")
        s = lax.axis_index("s")
        wid = c * NS + s

        def slot(b):
            return rows_v.at[pl.ds(b * CHUNK, CHUNK)]

        def gather_start(h_hbm, ch, b):
            pltpu.async_copy(h_hbm.at[idx_col.at[pl.ds(ch * CHUNK, CHUNK)]],
                             slot(b), gsem[b])

        def gather_wait(h_hbm, b):
            pltpu.make_async_copy(h_hbm.at[pl.ds(0, CHUNK)], slot(b),
                                  gsem[b]).wait()

        def scatter_start(ch, b):
            pltpu.async_copy(slot(b), acc_sh.at[idx_row.at[ch]], ssem[b],
                             add=True)

        def scatter_wait(b):
            pltpu.make_async_copy(slot(b), acc_sh.at[pl.ds(0, CHUNK)],
                                  ssem[b]).wait()

        def scale(ch, b):
            @pl.loop(0, CHUNK // 16)
            def _scale(g):
                a16 = adj_all[pl.ds(ch * CHUNK + g * 16, 16)]
                for i in range(16):
                    av = jnp.full((16,), a16[i], jnp.float32)
                    r = b * CHUNK + g * 16 + i
                    for j in range(d // 16):
                        sl = pl.ds(j * 16, 16)
                        rows_v[r, sl] = rows_v[r, sl] * av

        pltpu.sync_copy(row_hbm.at[wid], idx_row)
        pltpu.sync_copy(col_hbm.at[wid], idx_col)
        pltpu.sync_copy(adj_hbm.at[wid], adj_all)

        base = s * npt

        for part in range(nparts):
            h_hbm = h_parts[part]

            def work(ch, b):
                gather_wait(h_hbm, b)
                scale(ch, b)
                scatter_start(ch, b)

            @pl.loop(0, CHUNK)
            def _zero(r):
                for j in range(d // 16):
                    rows_v[r, pl.ds(j * 16, 16)] = jnp.zeros((16,),
                                                             jnp.float32)

            for k in range(pieces):
                pltpu.sync_copy(rows_v.at[pl.ds(0, CHUNK)],
                                acc_sh.at[pl.ds(base + k * CHUNK, CHUNK)])
            plsc.subcore_barrier()

            for b in range(PF):
                gather_start(h_hbm, b, b)
            for b in range(NB):
                tgt = b + PF
                if tgt >= NB:
                    scatter_wait(tgt % NB)
                gather_start(h_hbm, tgt, tgt % NB)
                work(b, b)

            @pl.loop(1, nouter - 1)
            def _groups(g0):
                for b in range(NB):
                    ch = g0 * NB + b
                    scatter_wait((b + PF) % NB)
                    gather_start(h_hbm, ch + PF, (b + PF) % NB)
                    work(ch, b)

            for b in range(NB):
                ch = nch - NB + b
                if b < NB - PF:
                    scatter_wait((b + PF) % NB)
                    gather_start(h_hbm, ch + PF, (b + PF) % NB)
                work(ch, b)
            for b in range(NB):
                scatter_wait(b)

            plsc.subcore_barrier()
            for k in range(pieces):
                pltpu.sync_copy(acc_sh.at[pl.ds(base + k * CHUNK, CHUNK)],
                                out_hbm.at[c, part,
                                           pl.ds(base + k * CHUNK, CHUNK)])
            if part + 1 < nparts:
                plsc.subcore_barrier()

    return pl.kernel(
        body,
        out_type=jax.ShapeDtypeStruct((NC, nparts, n, d), jnp.float32),
        compiler_params=pltpu.CompilerParams(use_tc_tiling_on_sc=False),
        mesh=plsc.VectorSubcoreMesh(core_axis_name="c", subcore_axis_name="s"),
        scratch_types=[
            pltpu.VMEM((nch, CHUNK), jnp.int32),
            pltpu.VMEM((epw,), jnp.int32),
            pltpu.VMEM((epw,), jnp.float32),
            pltpu.VMEM((NB * CHUNK, d), jnp.float32),
            pltpu.VMEM_SHARED((n, d), jnp.float32),
        ] + [pltpu.SemaphoreType.DMA] * (2 * NB),
    )


def _mm1(x, w):
    n, kdim = x.shape
    bm = 1000

    def kern(x_ref, w_ref, lo_ref, hi_ref):
        h = jnp.dot(x_ref[...], w_ref[...],
                    preferred_element_type=jnp.float32)
        lo_ref[...] = h[:, :D]
        hi_ref[...] = h[:, D:]

    return pl.pallas_call(
        kern,
        grid=(n // bm,),
        in_specs=[pl.BlockSpec((bm, kdim), lambda i: (i, 0)),
                  pl.BlockSpec((kdim, 2 * D), lambda i: (0, 0))],
        out_specs=[pl.BlockSpec((bm, D), lambda i: (i, 0)),
                   pl.BlockSpec((bm, D), lambda i: (i, 0))],
        out_shape=[jax.ShapeDtypeStruct((n, D), jnp.float32),
                   jax.ShapeDtypeStruct((n, D), jnp.float32)],
    )(x, w)


def _mid(p0lo, p1lo, p0hi, p1hi, b1, w2):
    n = p0lo.shape[0]
    kdim = w2.shape[0]
    m = w2.shape[1]
    bm = 1000

    def kern(a_ref, b_ref, c_ref, d_ref, b1_ref, w_ref, o_ref):
        lo = a_ref[...] + b_ref[...]
        hi = c_ref[...] + d_ref[...]
        a = jnp.maximum(jnp.concatenate([lo, hi], axis=1) + b1_ref[...], 0.0)
        o_ref[...] = jnp.dot(a, w_ref[...], preferred_element_type=jnp.float32)

    part = pl.BlockSpec((bm, D), lambda i: (i, 0))
    return pl.pallas_call(
        kern,
        grid=(n // bm,),
        in_specs=[part, part, part, part,
                  pl.BlockSpec((1, kdim), lambda i: (0, 0)),
                  pl.BlockSpec((kdim, m), lambda i: (0, 0))],
        out_specs=pl.BlockSpec((bm, m), lambda i: (i, 0)),
        out_shape=jax.ShapeDtypeStruct((n, m), jnp.float32),
    )(p0lo, p1lo, p0hi, p1hi, b1.reshape(1, kdim), w2)


def _post(q0, q1, b2p, nclass):
    n, dp = q0.shape
    bm = 1000

    def kern(q0_ref, q1_ref, b_ref, o_ref):
        z = q0_ref[...] + q1_ref[...] + b_ref[...]
        mask = lax.broadcasted_iota(jnp.int32, z.shape, 1) < nclass
        zm = jnp.where(mask, z, -jnp.inf)
        m = jnp.max(zm, axis=1, keepdims=True)
        ez = jnp.where(mask, jnp.exp(z - m), 0.0)
        lse = jnp.log(jnp.sum(ez, axis=1, keepdims=True))
        o_ref[...] = (z - m - lse)[:, :nclass]

    return pl.pallas_call(
        kern,
        grid=(n // bm,),
        in_specs=[pl.BlockSpec((bm, dp), lambda i: (i, 0)),
                  pl.BlockSpec((bm, dp), lambda i: (i, 0)),
                  pl.BlockSpec((1, dp), lambda i: (0, 0))],
        out_specs=pl.BlockSpec((bm, nclass), lambda i: (i, 0)),
        out_shape=jax.ShapeDtypeStruct((n, nclass), jnp.float32),
    )(q0, q1, b2p.reshape(1, dp))


def kernel(x, edge_index, adj_values, W1, b1, W2, b2):
    n, nfeat = x.shape
    e = edge_index.shape[1]
    nhid = W1.shape[1]
    nclass = W2.shape[1]
    d2 = 64

    nch = -(-e // (NW * CHUNK))
    nch = -(-nch // NB) * NB
    nch = max(nch, 2 * NB)
    ep = NW * nch * CHUNK
    row = jnp.pad(edge_index[0], (0, ep - e)).reshape(NW, nch, CHUNK)
    col = jnp.pad(edge_index[1], (0, ep - e)).reshape(NW, nch * CHUNK)
    adj = jnp.pad(adj_values, (0, ep - e)).reshape(NW, nch * CHUNK)

    w2p = jnp.pad(W2, ((0, 0), (0, d2 - nclass)))
    b2p = jnp.pad(b2, (0, d2 - nclass))

    npad = -(-n // (NS * CHUNK)) * NS * CHUNK

    h_lo, h_hi = _mm1(x, W1)
    p = _make_spmm(npad, nch, 2)(h_lo, h_hi, row, col, adj)
    h2 = _mid(p[0, 0, :n], p[1, 0, :n], p[0, 1, :n], p[1, 1, :n],
              b1, w2p)
    q = _make_spmm(npad, nch, 1)(h2, row, col, adj)
    return _post(q[0, 0, :n], q[1, 0, :n], b2p, nclass)


# speedup vs baseline: 1.2786x; 1.0872x over previous
import functools

import jax
import jax.numpy as jnp
from jax import lax
from jax.experimental import pallas as pl
from jax.experimental.pallas import tpu as pltpu
from jax.experimental.pallas import tpu_sc as plsc

NC = 2
NS = 16
NW = NC * NS
CHUNK = 128


NB = 5
PF = 3
D = 64


GRP = 8


def _make_spmm1(n, nchw):
    d = 128
    npt = n // NS
    ngrp = nchw // GRP
    npairs = nchw // (2 * GRP)
    assert n % NS == 0 and npt % CHUNK == 0
    assert nchw % (2 * GRP) == 0 and npairs >= 2
    pieces = npt // CHUNK
    gl = GRP * CHUNK

    def body(h_hbm, row_hbm, col_hbm, adj_hbm, out_hbm,
             idx_row_b, idx_col_b, adj_b, rows_v, acc_sh,
             gsem0, gsem1, ssem0, ssem1, isem0, isem1):
        gsem = (gsem0, gsem1)
        ssem = (ssem0, ssem1)
        isem = (isem0, isem1)
        c = lax.axis_index("c")
        s = lax.axis_index("s")
        wid = c * NS + s

        def slot(b):
            return rows_v.at[pl.ds(b * CHUNK, CHUNK)]

        def idx_load(g, ib):
            pltpu.async_copy(col_hbm.at[wid, g], idx_col_b.at[ib], isem[ib])
            pltpu.async_copy(row_hbm.at[wid, g], idx_row_b.at[ib], isem[ib])
            pltpu.async_copy(adj_hbm.at[wid, g], adj_b.at[ib], isem[ib])

        def idx_wait(ib):
            pltpu.make_async_copy(col_hbm.at[0, 0], idx_col_b.at[ib],
                                  isem[ib]).wait()
            pltpu.make_async_copy(row_hbm.at[0, 0], idx_row_b.at[ib],
                                  isem[ib]).wait()
            pltpu.make_async_copy(adj_hbm.at[0, 0], adj_b.at[ib],
                                  isem[ib]).wait()

        def gather_start(ib, k, b):
            pltpu.async_copy(
                h_hbm.at[idx_col_b.at[ib, pl.ds(k * CHUNK, CHUNK)]],
                slot(b), gsem[b])

        def gather_wait(b):
            pltpu.make_async_copy(h_hbm.at[pl.ds(0, CHUNK)], slot(b),
                                  gsem[b]).wait()

        def scatter_start(ib, k, b):
            pltpu.async_copy(slot(b), acc_sh.at[idx_row_b.at[ib, k]],
                             ssem[b], add=True)

        def scatter_wait(b):
            pltpu.make_async_copy(slot(b), acc_sh.at[pl.ds(0, CHUNK)],
                                  ssem[b]).wait()

        def scale(ib, k, b):
            @pl.loop(0, CHUNK // 16)
            def _scale(g2):
                a16 = adj_b[ib, pl.ds(k * CHUNK + g2 * 16, 16)]
                for i in range(16):
                    av = jnp.full((16,), a16[i], jnp.float32)
                    r = b * CHUNK + g2 * 16 + i
                    for j in range(d // 16):
                        sl = pl.ds(j * 16, 16)
                        rows_v[r, sl] = rows_v[r, sl] * av

        @pl.loop(0, CHUNK)
        def _zero(r):
            for j in range(d // 16):
                rows_v[r, pl.ds(j * 16, 16)] = jnp.zeros((16,), jnp.float32)

        base = s * npt
        for k in range(pieces):
            pltpu.sync_copy(rows_v.at[pl.ds(0, CHUNK)],
                            acc_sh.at[pl.ds(base + k * CHUNK, CHUNK)])
        plsc.subcore_barrier()

        def step(gp, gg, k, first, last):
            b = (gg * GRP + k) % 2
            ib = gg
            if not (first and gg == 0 and k == 0):
                scatter_wait((b + 1) % 2)
            if k == 0 and not (last and gg == 1):
                idx_load(gp * 2 + gg + 1, (gg + 1) % 2)
            if not (last and gg == 1 and k == GRP - 1):
                k2 = (k + 1) % GRP
                ib2 = ib if k + 1 < GRP else (gg + 1) % 2
                if k + 1 == GRP:
                    idx_wait(ib2)
                gather_start(ib2, k2, (b + 1) % 2)
            gather_wait(b)
            scale(ib, k, b)
            scatter_start(ib, k, b)

        idx_load(0, 0)
        idx_wait(0)
        gather_start(0, 0, 0)
        for gg in range(2):
            for k in range(GRP):
                step(0, gg, k, True, npairs == 1)

        @pl.loop(1, npairs - 1)
        def _pairs(gp):
            for gg in range(2):
                for k in range(GRP):
                    step(gp, gg, k, False, False)

        for gg in range(2):
            for k in range(GRP):
                step(npairs - 1, gg, k, False, True)
        scatter_wait((nchw - 1) % 2)

        plsc.subcore_barrier()
        for k in range(pieces):
            pltpu.sync_copy(acc_sh.at[pl.ds(base + k * CHUNK, CHUNK)],
                            out_hbm.at[c, pl.ds(base + k * CHUNK, CHUNK)])

    return pl.kernel(
        body,
        out_type=jax.ShapeDtypeStruct((NC, n, d), jnp.float32),
        compiler_params=pltpu.CompilerParams(use_tc_tiling_on_sc=False),
        mesh=plsc.VectorSubcoreMesh(core_axis_name="c", subcore_axis_name="s"),
        scratch_types=[
            pltpu.VMEM((2, GRP, CHUNK), jnp.int32),
            pltpu.VMEM((2, gl), jnp.int32),
            pltpu.VMEM((2, gl), jnp.float32),
            pltpu.VMEM((2 * CHUNK, d), jnp.float32),
            pltpu.VMEM_SHARED((n, d), jnp.float32),
        ] + [pltpu.SemaphoreType.DMA] * 6,
    )


def _make_spmm(n, nch, nparts):
    d = D
    epw = nch * CHUNK
    npt = n // NS
    assert n % NS == 0 and npt % CHUNK == 0 and nch % NB == 0 and nch >= 2 * NB
    pieces = npt // CHUNK
    nouter = nch // NB

    def body(*args):
        h_parts = args[:nparts]
        (row_hbm, col_hbm, adj_hbm, out_hbm,
         idx_row, idx_col, adj_all, rows_v, acc_sh) = args[nparts:nparts + 9]
        sems = args[nparts + 9:]
        gsem = sems[:NB]
        ssem = sems[NB:2 * NB]
        c = lax.axis_index("c")
        s = lax.axis_index("s")
        wid = c * NS + s

        def slot(b):
            return rows_v.at[pl.ds(b * CHUNK, CHUNK)]

        def gather_start(h_hbm, ch, b):
            pltpu.async_copy(h_hbm.at[idx_col.at[pl.ds(ch * CHUNK, CHUNK)]],
                             slot(b), gsem[b])

        def gather_wait(h_hbm, b):
            pltpu.make_async_copy(h_hbm.at[pl.ds(0, CHUNK)], slot(b),
                                  gsem[b]).wait()

        def scatter_start(ch, b):
            pltpu.async_copy(slot(b), acc_sh.at[idx_row.at[ch]], ssem[b],
                             add=True)

        def scatter_wait(b):
            pltpu.make_async_copy(slot(b), acc_sh.at[pl.ds(0, CHUNK)],
                                  ssem[b]).wait()

        def scale(ch, b):
            @pl.loop(0, CHUNK // 16)
            def _scale(g):
                a16 = adj_all[pl.ds(ch * CHUNK + g * 16, 16)]
                for i in range(16):
                    av = jnp.full((16,), a16[i], jnp.float32)
                    r = b * CHUNK + g * 16 + i
                    for j in range(d // 16):
                        sl = pl.ds(j * 16, 16)
                        rows_v[r, sl] = rows_v[r, sl] * av

        pltpu.sync_copy(row_hbm.at[wid], idx_row)
        pltpu.sync_copy(col_hbm.at[wid], idx_col)
        pltpu.sync_copy(adj_hbm.at[wid], adj_all)

        base = s * npt

        for part in range(nparts):
            h_hbm = h_parts[part]

            def work(ch, b):
                gather_wait(h_hbm, b)
                scale(ch, b)
                scatter_start(ch, b)

            @pl.loop(0, CHUNK)
            def _zero(r):
                for j in range(d // 16):
                    rows_v[r, pl.ds(j * 16, 16)] = jnp.zeros((16,),
                                                             jnp.float32)

            for k in range(pieces):
                pltpu.sync_copy(rows_v.at[pl.ds(0, CHUNK)],
                                acc_sh.at[pl.ds(base + k * CHUNK, CHUNK)])
            plsc.subcore_barrier()

            for b in range(PF):
                gather_start(h_hbm, b, b)
            for b in range(NB):
                tgt = b + PF
                if tgt >= NB:
                    scatter_wait(tgt % NB)
                gather_start(h_hbm, tgt, tgt % NB)
                work(b, b)

            @pl.loop(1, nouter - 1)
            def _groups(g0):
                for b in range(NB):
                    ch = g0 * NB + b
                    scatter_wait((b + PF) % NB)
                    gather_start(h_hbm, ch + PF, (b + PF) % NB)
                    work(ch, b)

            for b in range(NB):
                ch = nch - NB + b
                if b < NB - PF:
                    scatter_wait((b + PF) % NB)
                    gather_start(h_hbm, ch + PF, (b + PF) % NB)
                work(ch, b)
            for b in range(NB):
                scatter_wait(b)

            plsc.subcore_barrier()
            for k in range(pieces):
                pltpu.sync_copy(acc_sh.at[pl.ds(base + k * CHUNK, CHUNK)],
                                out_hbm.at[c, part,
                                           pl.ds(base + k * CHUNK, CHUNK)])
            if part + 1 < nparts:
                plsc.subcore_barrier()

    return pl.kernel(
        body,
        out_type=jax.ShapeDtypeStruct((NC, nparts, n, d), jnp.float32),
        compiler_params=pltpu.CompilerParams(use_tc_tiling_on_sc=False),
        mesh=plsc.VectorSubcoreMesh(core_axis_name="c", subcore_axis_name="s"),
        scratch_types=[
            pltpu.VMEM((nch, CHUNK), jnp.int32),
            pltpu.VMEM((epw,), jnp.int32),
            pltpu.VMEM((epw,), jnp.float32),
            pltpu.VMEM((NB * CHUNK, d), jnp.float32),
            pltpu.VMEM_SHARED((n, d), jnp.float32),
        ] + [pltpu.SemaphoreType.DMA] * (2 * NB),
    )


def _mm1(x, w):
    n, kdim = x.shape
    m = w.shape[1]
    bm = 1000

    def kern(x_ref, w_ref, o_ref):
        o_ref[...] = jnp.dot(x_ref[...], w_ref[...],
                             preferred_element_type=jnp.float32)

    return pl.pallas_call(
        kern,
        grid=(n // bm,),
        in_specs=[pl.BlockSpec((bm, kdim), lambda i: (i, 0)),
                  pl.BlockSpec((kdim, m), lambda i: (0, 0))],
        out_specs=pl.BlockSpec((bm, m), lambda i: (i, 0)),
        out_shape=jax.ShapeDtypeStruct((n, m), jnp.float32),
    )(x, w)


def _mid(p0, p1, b1, w2):
    n, kdim = p0.shape
    m = w2.shape[1]
    bm = 1000

    def kern(p0_ref, p1_ref, b1_ref, w_ref, o_ref):
        a = jnp.maximum(p0_ref[...] + p1_ref[...] + b1_ref[...], 0.0)
        o_ref[...] = jnp.dot(a, w_ref[...], preferred_element_type=jnp.float32)

    return pl.pallas_call(
        kern,
        grid=(n // bm,),
        in_specs=[pl.BlockSpec((bm, kdim), lambda i: (i, 0)),
                  pl.BlockSpec((bm, kdim), lambda i: (i, 0)),
                  pl.BlockSpec((1, kdim), lambda i: (0, 0)),
                  pl.BlockSpec((kdim, m), lambda i: (0, 0))],
        out_specs=pl.BlockSpec((bm, m), lambda i: (i, 0)),
        out_shape=jax.ShapeDtypeStruct((n, m), jnp.float32),
    )(p0, p1, b1.reshape(1, kdim), w2)


def _post(q0, q1, b2p, nclass):
    n, dp = q0.shape
    bm = 1000

    def kern(q0_ref, q1_ref, b_ref, o_ref):
        z = q0_ref[...] + q1_ref[...] + b_ref[...]
        mask = lax.broadcasted_iota(jnp.int32, z.shape, 1) < nclass
        zm = jnp.where(mask, z, -jnp.inf)
        m = jnp.max(zm, axis=1, keepdims=True)
        ez = jnp.where(mask, jnp.exp(z - m), 0.0)
        lse = jnp.log(jnp.sum(ez, axis=1, keepdims=True))
        o_ref[...] = (z - m - lse)[:, :nclass]

    return pl.pallas_call(
        kern,
        grid=(n // bm,),
        in_specs=[pl.BlockSpec((bm, dp), lambda i: (i, 0)),
                  pl.BlockSpec((bm, dp), lambda i: (i, 0)),
                  pl.BlockSpec((1, dp), lambda i: (0, 0))],
        out_specs=pl.BlockSpec((bm, nclass), lambda i: (i, 0)),
        out_shape=jax.ShapeDtypeStruct((n, nclass), jnp.float32),
    )(q0, q1, b2p.reshape(1, dp))


def kernel(x, edge_index, adj_values, W1, b1, W2, b2):
    n, nfeat = x.shape
    e = edge_index.shape[1]
    nhid = W1.shape[1]
    nclass = W2.shape[1]
    d2 = 64

    nch = -(-e // (NW * CHUNK))
    nch = -(-nch // NB) * NB
    nch = max(nch, 2 * NB)
    ep = NW * nch * CHUNK
    row = jnp.pad(edge_index[0], (0, ep - e)).reshape(NW, nch, CHUNK)
    col = jnp.pad(edge_index[1], (0, ep - e)).reshape(NW, nch * CHUNK)
    adj = jnp.pad(adj_values, (0, ep - e)).reshape(NW, nch * CHUNK)

    nchw = -(-e // (NW * CHUNK))
    nchw = max(-(-nchw // (2 * GRP)) * 2 * GRP, 4 * GRP)
    ep1 = NW * nchw * CHUNK
    ngrp = nchw // GRP
    row1 = jnp.pad(edge_index[0], (0, ep1 - e)).reshape(NW, ngrp, GRP, CHUNK)
    col1 = jnp.pad(edge_index[1], (0, ep1 - e)).reshape(NW, ngrp, GRP * CHUNK)
    adj1 = jnp.pad(adj_values, (0, ep1 - e)).reshape(NW, ngrp, GRP * CHUNK)

    w2p = jnp.pad(W2, ((0, 0), (0, d2 - nclass)))
    b2p = jnp.pad(b2, (0, d2 - nclass))

    npad = -(-n // (NS * CHUNK)) * NS * CHUNK

    h = _mm1(x, W1)
    p = _make_spmm1(npad, nchw)(h, row1, col1, adj1)
    h2 = _mid(p[0, :n], p[1, :n], b1, w2p)
    q = _make_spmm(npad, nch, 1)(h2, row, col, adj)
    return _post(q[0, 0, :n], q[1, 0, :n], b2p, nclass)


# speedup vs baseline: 1.3742x; 1.0747x over previous
import functools

import jax
import jax.numpy as jnp
from jax import lax
from jax.experimental import pallas as pl
from jax.experimental.pallas import tpu as pltpu
from jax.experimental.pallas import tpu_sc as plsc

NC = 2
NS = 16
NW = NC * NS
CHUNK = 128


NB = 5
PF = 3
D = 64


GRP = 8


def _make_spmm1(n, nchw):
    d = 128
    npt = n // NS
    ngrp = nchw // GRP
    npairs = nchw // (2 * GRP)
    assert n % NS == 0 and npt % CHUNK == 0
    assert nchw % (2 * GRP) == 0 and npairs >= 2
    pieces = npt // CHUNK
    gl = GRP * CHUNK

    def body(h_hbm, row_hbm, col_hbm, adj_hbm, out_hbm,
             idx_row_b, idx_col_b, adj_b, rows_v, acc_sh,
             gsem0, gsem1, ssem0, ssem1, isem0, isem1):
        gsem = (gsem0, gsem1)
        ssem = (ssem0, ssem1)
        isem = (isem0, isem1)
        c = lax.axis_index("c")
        s = lax.axis_index("s")
        wid = c * NS + s

        def slot(b):
            return rows_v.at[pl.ds(b * CHUNK, CHUNK)]

        def idx_load(g, ib):
            pltpu.async_copy(col_hbm.at[wid, g], idx_col_b.at[ib], isem[ib])
            pltpu.async_copy(row_hbm.at[wid, g], idx_row_b.at[ib], isem[ib])
            pltpu.async_copy(adj_hbm.at[wid, g], adj_b.at[ib], isem[ib])

        def idx_wait(ib):
            pltpu.make_async_copy(col_hbm.at[0, 0], idx_col_b.at[ib],
                                  isem[ib]).wait()
            pltpu.make_async_copy(row_hbm.at[0, 0], idx_row_b.at[ib],
                                  isem[ib]).wait()
            pltpu.make_async_copy(adj_hbm.at[0, 0], adj_b.at[ib],
                                  isem[ib]).wait()

        def gather_start(ib, k, b):
            pltpu.async_copy(
                h_hbm.at[idx_col_b.at[ib, pl.ds(k * CHUNK, CHUNK)]],
                slot(b), gsem[b])

        def gather_wait(b):
            pltpu.make_async_copy(h_hbm.at[pl.ds(0, CHUNK)], slot(b),
                                  gsem[b]).wait()

        def scatter_start(ib, k, b):
            pltpu.async_copy(slot(b), acc_sh.at[idx_row_b.at[ib, k]],
                             ssem[b], add=True)

        def scatter_wait(b):
            pltpu.make_async_copy(slot(b), acc_sh.at[pl.ds(0, CHUNK)],
                                  ssem[b]).wait()

        def scale(ib, k, b):
            @pl.loop(0, CHUNK // 16)
            def _scale(g2):
                a16 = adj_b[ib, pl.ds(k * CHUNK + g2 * 16, 16)]
                for i in range(16):
                    av = jnp.full((16,), a16[i], jnp.float32)
                    r = b * CHUNK + g2 * 16 + i
                    for j in range(d // 16):
                        sl = pl.ds(j * 16, 16)
                        rows_v[r, sl] = rows_v[r, sl] * av

        @pl.loop(0, CHUNK)
        def _zero(r):
            for j in range(d // 16):
                rows_v[r, pl.ds(j * 16, 16)] = jnp.zeros((16,), jnp.float32)

        base = s * npt
        for k in range(pieces):
            pltpu.sync_copy(rows_v.at[pl.ds(0, CHUNK)],
                            acc_sh.at[pl.ds(base + k * CHUNK, CHUNK)])
        plsc.subcore_barrier()

        def step(gp, gg, k, first, last):
            b = (gg * GRP + k) % 2
            ib = gg
            if not (first and gg == 0 and k == 0):
                scatter_wait((b + 1) % 2)
            if k == 0 and not (last and gg == 1):
                idx_load(gp * 2 + gg + 1, (gg + 1) % 2)
            if not (last and gg == 1 and k == GRP - 1):
                k2 = (k + 1) % GRP
                ib2 = ib if k + 1 < GRP else (gg + 1) % 2
                if k + 1 == GRP:
                    idx_wait(ib2)
                gather_start(ib2, k2, (b + 1) % 2)
            gather_wait(b)
            scale(ib, k, b)
            scatter_start(ib, k, b)

        idx_load(0, 0)
        idx_wait(0)
        gather_start(0, 0, 0)
        for gg in range(2):
            for k in range(GRP):
                step(0, gg, k, True, npairs == 1)

        @pl.loop(1, npairs - 1)
        def _pairs(gp):
            for gg in range(2):
                for k in range(GRP):
                    step(gp, gg, k, False, False)

        for gg in range(2):
            for k in range(GRP):
                step(npairs - 1, gg, k, False, True)
        scatter_wait((nchw - 1) % 2)

        plsc.subcore_barrier()
        for k in range(pieces):
            pltpu.sync_copy(acc_sh.at[pl.ds(base + k * CHUNK, CHUNK)],
                            out_hbm.at[c, pl.ds(base + k * CHUNK, CHUNK)])

    return pl.kernel(
        body,
        out_type=jax.ShapeDtypeStruct((NC, n, d), jnp.float32),
        compiler_params=pltpu.CompilerParams(use_tc_tiling_on_sc=False),
        mesh=plsc.VectorSubcoreMesh(core_axis_name="c", subcore_axis_name="s"),
        scratch_types=[
            pltpu.VMEM((2, GRP, CHUNK), jnp.int32),
            pltpu.VMEM((2, gl), jnp.int32),
            pltpu.VMEM((2, gl), jnp.float32),
            pltpu.VMEM((2 * CHUNK, d), jnp.float32),
            pltpu.VMEM_SHARED((n, d), jnp.float32),
        ] + [pltpu.SemaphoreType.DMA] * 6,
    )


def _make_spmm(n, nch, nparts, d=D):
    epw = nch * CHUNK
    npt = n // NS
    assert n % NS == 0 and npt % CHUNK == 0 and nch % NB == 0 and nch >= 2 * NB
    pieces = npt // CHUNK
    nouter = nch // NB

    def body(*args):
        h_parts = args[:nparts]
        (row_hbm, col_hbm, adj_hbm, out_hbm,
         idx_row, idx_col, adj_all, rows_v, acc_sh) = args[nparts:nparts + 9]
        sems = args[nparts + 9:]
        gsem = sems[:NB]
        ssem = sems[NB:2 * NB]
        c = lax.axis_index("c")
        s = lax.axis_index("s")
        wid = c * NS + s

        def slot(b):
            return rows_v.at[pl.ds(b * CHUNK, CHUNK)]

        def gather_start(h_hbm, ch, b):
            pltpu.async_copy(h_hbm.at[idx_col.at[pl.ds(ch * CHUNK, CHUNK)]],
                             slot(b), gsem[b])

        def gather_wait(h_hbm, b):
            pltpu.make_async_copy(h_hbm.at[pl.ds(0, CHUNK)], slot(b),
                                  gsem[b]).wait()

        def scatter_start(ch, b):
            pltpu.async_copy(slot(b), acc_sh.at[idx_row.at[ch]], ssem[b],
                             add=True)

        def scatter_wait(b):
            pltpu.make_async_copy(slot(b), acc_sh.at[pl.ds(0, CHUNK)],
                                  ssem[b]).wait()

        def scale(ch, b):
            @pl.loop(0, CHUNK // 16)
            def _scale(g):
                a16 = adj_all[pl.ds(ch * CHUNK + g * 16, 16)]
                for i in range(16):
                    av = jnp.full((16,), a16[i], jnp.float32)
                    r = b * CHUNK + g * 16 + i
                    for j in range(d // 16):
                        sl = pl.ds(j * 16, 16)
                        rows_v[r, sl] = rows_v[r, sl] * av

        pltpu.sync_copy(row_hbm.at[wid], idx_row)
        pltpu.sync_copy(col_hbm.at[wid], idx_col)
        pltpu.sync_copy(adj_hbm.at[wid], adj_all)

        base = s * npt

        for part in range(nparts):
            h_hbm = h_parts[part]

            def work(ch, b):
                gather_wait(h_hbm, b)
                scale(ch, b)
                scatter_start(ch, b)

            @pl.loop(0, CHUNK)
            def _zero(r):
                for j in range(d // 16):
                    rows_v[r, pl.ds(j * 16, 16)] = jnp.zeros((16,),
                                                             jnp.float32)

            for k in range(pieces):
                pltpu.sync_copy(rows_v.at[pl.ds(0, CHUNK)],
                                acc_sh.at[pl.ds(base + k * CHUNK, CHUNK)])
            plsc.subcore_barrier()

            for b in range(PF):
                gather_start(h_hbm, b, b)
            for b in range(NB):
                tgt = b + PF
                if tgt >= NB:
                    scatter_wait(tgt % NB)
                gather_start(h_hbm, tgt, tgt % NB)
                work(b, b)

            @pl.loop(1, nouter - 1)
            def _groups(g0):
                for b in range(NB):
                    ch = g0 * NB + b
                    scatter_wait((b + PF) % NB)
                    gather_start(h_hbm, ch + PF, (b + PF) % NB)
                    work(ch, b)

            for b in range(NB):
                ch = nch - NB + b
                if b < NB - PF:
                    scatter_wait((b + PF) % NB)
                    gather_start(h_hbm, ch + PF, (b + PF) % NB)
                work(ch, b)
            for b in range(NB):
                scatter_wait(b)

            plsc.subcore_barrier()
            for k in range(pieces):
                pltpu.sync_copy(acc_sh.at[pl.ds(base + k * CHUNK, CHUNK)],
                                out_hbm.at[c, part,
                                           pl.ds(base + k * CHUNK, CHUNK)])
            if part + 1 < nparts:
                plsc.subcore_barrier()

    return pl.kernel(
        body,
        out_type=jax.ShapeDtypeStruct((NC, nparts, n, d), jnp.float32),
        compiler_params=pltpu.CompilerParams(use_tc_tiling_on_sc=False),
        mesh=plsc.VectorSubcoreMesh(core_axis_name="c", subcore_axis_name="s"),
        scratch_types=[
            pltpu.VMEM((nch, CHUNK), jnp.int32),
            pltpu.VMEM((epw,), jnp.int32),
            pltpu.VMEM((epw,), jnp.float32),
            pltpu.VMEM((NB * CHUNK, d), jnp.float32),
            pltpu.VMEM_SHARED((n, d), jnp.float32),
        ] + [pltpu.SemaphoreType.DMA] * (2 * NB),
    )


def _mm1(x, w):
    n, kdim = x.shape
    m = w.shape[1]
    bm = 1000

    def kern(x_ref, w_ref, o_ref):
        o_ref[...] = jnp.dot(x_ref[...], w_ref[...],
                             preferred_element_type=jnp.float32)

    return pl.pallas_call(
        kern,
        grid=(n // bm,),
        in_specs=[pl.BlockSpec((bm, kdim), lambda i: (i, 0)),
                  pl.BlockSpec((kdim, m), lambda i: (0, 0))],
        out_specs=pl.BlockSpec((bm, m), lambda i: (i, 0)),
        out_shape=jax.ShapeDtypeStruct((n, m), jnp.float32),
    )(x, w)


def _mid(p0, p1, b1, w2):
    n, kdim = p0.shape
    m = w2.shape[1]
    bm = 1000

    def kern(p0_ref, p1_ref, b1_ref, w_ref, o_ref):
        a = jnp.maximum(p0_ref[...] + p1_ref[...] + b1_ref[...], 0.0)
        o_ref[...] = jnp.dot(a, w_ref[...], preferred_element_type=jnp.float32)

    return pl.pallas_call(
        kern,
        grid=(n // bm,),
        in_specs=[pl.BlockSpec((bm, kdim), lambda i: (i, 0)),
                  pl.BlockSpec((bm, kdim), lambda i: (i, 0)),
                  pl.BlockSpec((1, kdim), lambda i: (0, 0)),
                  pl.BlockSpec((kdim, m), lambda i: (0, 0))],
        out_specs=pl.BlockSpec((bm, m), lambda i: (i, 0)),
        out_shape=jax.ShapeDtypeStruct((n, m), jnp.float32),
    )(p0, p1, b1.reshape(1, kdim), w2)


def _post(q0, q1, b2p, nclass):
    n, dp = q0.shape
    bm = 1000

    def kern(q0_ref, q1_ref, b_ref, o_ref):
        z = q0_ref[...] + q1_ref[...] + b_ref[...]
        mask = lax.broadcasted_iota(jnp.int32, z.shape, 1) < nclass
        zm = jnp.where(mask, z, -jnp.inf)
        m = jnp.max(zm, axis=1, keepdims=True)
        ez = jnp.where(mask, jnp.exp(z - m), 0.0)
        lse = jnp.log(jnp.sum(ez, axis=1, keepdims=True))
        o_ref[...] = (z - m - lse)[:, :nclass]

    return pl.pallas_call(
        kern,
        grid=(n // bm,),
        in_specs=[pl.BlockSpec((bm, dp), lambda i: (i, 0)),
                  pl.BlockSpec((bm, dp), lambda i: (i, 0)),
                  pl.BlockSpec((1, dp), lambda i: (0, 0))],
        out_specs=pl.BlockSpec((bm, nclass), lambda i: (i, 0)),
        out_shape=jax.ShapeDtypeStruct((n, nclass), jnp.float32),
    )(q0, q1, b2p.reshape(1, dp))


def kernel(x, edge_index, adj_values, W1, b1, W2, b2):
    n, nfeat = x.shape
    e = edge_index.shape[1]
    nhid = W1.shape[1]
    nclass = W2.shape[1]
    d2 = 48

    nch = -(-e // (NW * CHUNK))
    nch = -(-nch // NB) * NB
    nch = max(nch, 2 * NB)
    ep = NW * nch * CHUNK
    row = jnp.pad(edge_index[0], (0, ep - e)).reshape(NW, nch, CHUNK)
    col = jnp.pad(edge_index[1], (0, ep - e)).reshape(NW, nch * CHUNK)
    adj = jnp.pad(adj_values, (0, ep - e)).reshape(NW, nch * CHUNK)

    nchw = -(-e // (NW * CHUNK))
    nchw = max(-(-nchw // (2 * GRP)) * 2 * GRP, 4 * GRP)
    ep1 = NW * nchw * CHUNK
    ngrp = nchw // GRP
    row1 = jnp.pad(edge_index[0], (0, ep1 - e)).reshape(NW, ngrp, GRP, CHUNK)
    col1 = jnp.pad(edge_index[1], (0, ep1 - e)).reshape(NW, ngrp, GRP * CHUNK)
    adj1 = jnp.pad(adj_values, (0, ep1 - e)).reshape(NW, ngrp, GRP * CHUNK)

    w2p = jnp.pad(W2, ((0, 0), (0, d2 - nclass)))
    b2p = jnp.pad(b2, (0, d2 - nclass))

    npad = -(-n // (NS * CHUNK)) * NS * CHUNK

    h = _mm1(x, W1)
    p = _make_spmm1(npad, nchw)(h, row1, col1, adj1)
    h2 = _mid(p[0, :n], p[1, :n], b1, w2p)
    q = _make_spmm(npad, nch, 1, d2)(h2, row, col, adj)
    return _post(q[0, 0, :n], q[1, 0, :n], b2p, nclass)
